# Initial kernel scaffold; baseline (speedup 1.0000x reference)
import jax
import jax.numpy as jnp
from jax.experimental import pallas as pl


def kernel(inputs):
    raise NotImplementedError("write your pallas kernel here")


# speedup vs baseline: 1.3004x; 1.3004x over previous
import functools

import jax
import jax.numpy as jnp
from jax import lax
from jax.experimental import pallas as pl
from jax.experimental.pallas import tpu as pltpu
from jax.experimental.pallas import tpu_sc as plsc

_B = 1024
_S = 50
_D = 1000
_NW = 32
_RPW = _B // _NW
_GRP = _RPW // 16
_IN_W = _RPW * _S
_OUT_W = _RPW * _D

_mesh = plsc.VectorSubcoreMesh(core_axis_name="c", subcore_axis_name="s")


@functools.partial(
    pl.kernel,
    mesh=_mesh,
    out_type=jax.ShapeDtypeStruct((_B * _D,), jnp.float32),
    compiler_params=pltpu.CompilerParams(needs_layout_passes=False),
    scratch_types=[
        pltpu.VMEM((_IN_W,), jnp.float32),
        pltpu.VMEM((_OUT_W,), jnp.float32),
    ],
)
def _hist_kernel(in_hbm, out_hbm, in_v, out_v):
    wid = lax.axis_index("s") * 2 + lax.axis_index("c")

    pltpu.sync_copy(in_hbm.at[pl.ds(wid * _IN_W, _IN_W)], in_v)

    zeros = jnp.zeros((16,), jnp.float32)

    def zbody(i, carry):
        base = pl.multiple_of(i * 128, 128)
        for k in range(8):
            out_v[pl.ds(base + k * 16, 16)] = zeros
        return carry

    lax.fori_loop(0, _OUT_W // 128, zbody, 0, unroll=False)

    lanes = lax.iota(jnp.int32, 16)
    ones = jnp.ones((16,), jnp.float32)

    for g in range(_GRP):
        row_base = lanes * _D + g * 16 * _D
        src_base = lanes * _S + g * 16 * _S

        def cbody(c, carry, row_base=row_base, src_base=src_base):
            vals = plsc.load_gather(in_v, [src_base + c])
            idx = row_base + vals.astype(jnp.int32)
            plsc.addupdate_scatter(out_v, [idx], ones)
            return carry

        lax.fori_loop(0, _S, cbody, 0, unroll=False)

    pltpu.sync_copy(out_v, out_hbm.at[pl.ds(wid * _OUT_W, _OUT_W)])


def kernel(inputs):
    flat = jnp.reshape(inputs, (_B * _S,))
    out = _hist_kernel(flat)
    return jnp.reshape(out, (_B, _D))
